# baseline (speedup 1.0000x reference)
import functools

import jax
import jax.numpy as jnp
import numpy as np
from jax.experimental import pallas as pl

B, S, H = 2, 4096, 768
NH = 8
D = H // NH
MAX_SELECTED = 2048
TOPK = min(MAX_SELECTED, S)

TP = 1024
TS = 512
TT = 1024

_PREC = jax.lax.Precision.DEFAULT


def _proj_kernel(x_ref, w_ref, o_ref):
    o_ref[...] = jax.lax.dot(
        x_ref[0], w_ref[0], precision=_PREC,
        preferred_element_type=jnp.float32)[None]


def _score_kernel(q_ref, k_ref, w_ref, o_ref):
    ti = pl.program_id(2)

    @pl.when(ti == 0)
    def _init():
        o_ref[...] = jnp.zeros_like(o_ref)

    q = q_ref[0]
    k = k_ref[0]
    acc = jnp.zeros((1, TS), jnp.float32)
    for h in range(NH):
        qh = q[:, h * D:(h + 1) * D]
        kh = k[:, h * D:(h + 1) * D]
        p = jax.lax.dot_general(
            kh, qh, (((1,), (1,)), ((), ())),
            precision=_PREC, preferred_element_type=jnp.float32)
        acc = acc + jnp.sum(jnp.maximum(p, 0.0), axis=0,
                            keepdims=True) * w_ref[0, h]
    o_ref[...] += acc[None, None]


def _topk_kernel(s_ref, t_ref, mask_ref, out_ref):
    scale = jnp.exp(-t_ref[0, 0])
    raw = s_ref[...]
    out_ref[...] = raw * scale

    i = jax.lax.bitcast_convert_type(raw, jnp.int32)
    key = jnp.where(i >= 0, i, i ^ jnp.int32(0x7FFFFFFF))
    imin = jnp.int32(-2147483648)
    uk = key ^ imin

    k_sel = jnp.int32(TOPK)
    prefix = jnp.zeros((B, 1), jnp.int32)
    count_above = jnp.zeros((B, 1), jnp.int32)
    for b in range(31, -1, -1):
        bit = imin if b == 31 else jnp.int32(1 << b)
        hmask = jnp.int32(-(1 << b))
        cand = prefix | bit
        match = ((uk ^ cand) & hmask) == 0
        c = jnp.sum(match.astype(jnp.int32), axis=-1, keepdims=True)
        take = (count_above + c) >= k_sel
        prefix = jnp.where(take, cand, prefix)
        count_above = jnp.where(take, count_above, count_above + c)

    t_key = prefix ^ imin
    gt = key > t_key
    eq = key == t_key
    needed = k_sel - count_above

    idx = jax.lax.broadcasted_iota(jnp.int32, (B, S), 1)
    lo = jnp.zeros((B, 1), jnp.int32)
    hi = jnp.full((B, 1), S - 1, jnp.int32)
    for _ in range(12):
        mid = (lo + hi) // 2
        cnt = jnp.sum((eq & (idx <= mid)).astype(jnp.int32),
                      axis=-1, keepdims=True)
        ok = cnt >= needed
        hi = jnp.where(ok, mid, hi)
        lo = jnp.where(ok, lo, mid + 1)

    mask_ref[...] = (gt | (eq & (idx <= lo))).astype(jnp.int32)


def kernel(query_states, key_states, Wq, Wk, head_weights, temperature_param):
    x = jnp.concatenate([query_states, key_states], axis=0)
    w = jnp.stack([Wq.T, Wk.T], axis=0)

    proj = pl.pallas_call(
        _proj_kernel,
        grid=(2 * B, S // TP),
        in_specs=[
            pl.BlockSpec((1, TP, H), lambda i, j: (i, j, 0)),
            pl.BlockSpec((1, H, H), lambda i, j: (i // B, 0, 0)),
        ],
        out_specs=pl.BlockSpec((1, TP, H), lambda i, j: (i, j, 0)),
        out_shape=jax.ShapeDtypeStruct((2 * B, S, H), jnp.float32),
    )(x, w)
    q_proj, k_proj = proj[:B], proj[B:]

    hw = head_weights.reshape(1, NH).astype(jnp.float32)

    raw_scores = pl.pallas_call(
        _score_kernel,
        grid=(B, S // TS, S // TT),
        in_specs=[
            pl.BlockSpec((1, TS, H), lambda b, si, ti: (b, si, 0)),
            pl.BlockSpec((1, TT, H), lambda b, si, ti: (b, ti, 0)),
            pl.BlockSpec((1, NH), lambda b, si, ti: (0, 0)),
        ],
        out_specs=pl.BlockSpec((1, 1, 1, TS), lambda b, si, ti: (b, si, 0, 0)),
        out_shape=jax.ShapeDtypeStruct((B, S // TS, 1, TS), jnp.float32),
    )(q_proj, k_proj, hw)
    raw_scores = raw_scores.reshape(B, S)

    temp = temperature_param.reshape(1, 1).astype(jnp.float32)
    mask_i32, scores = pl.pallas_call(
        _topk_kernel,
        grid=(1,),
        in_specs=[
            pl.BlockSpec((B, S), lambda i: (0, 0)),
            pl.BlockSpec((1, 1), lambda i: (0, 0)),
        ],
        out_specs=[
            pl.BlockSpec((B, S), lambda i: (0, 0)),
            pl.BlockSpec((B, S), lambda i: (0, 0)),
        ],
        out_shape=[
            jax.ShapeDtypeStruct((B, S), jnp.int32),
            jax.ShapeDtypeStruct((B, S), jnp.float32),
        ],
    )(raw_scores, temp)

    return (mask_i32.astype(jnp.bool_), scores)


# speedup vs baseline: 1.4013x; 1.4013x over previous
import jax
import jax.numpy as jnp
from jax.experimental import pallas as pl
from jax.experimental.pallas import tpu as pltpu

B, S, H = 2, 4096, 768
NH = 8
D = H // NH
TOPK = min(2048, S)

TS = 512
TT = 1024
NS = S // TS
NT = S // TT

_PREC = jax.lax.Precision.DEFAULT


def _topk_mask(scores):
    i = jax.lax.bitcast_convert_type(scores, jnp.int32)
    key = jnp.where(i >= 0, i, i ^ jnp.int32(0x7FFFFFFF))
    imin = jnp.int32(-2147483648)
    uk = key ^ imin

    k_sel = jnp.int32(TOPK)
    prefix = jnp.zeros((B, 1), jnp.int32)
    count_above = jnp.zeros((B, 1), jnp.int32)
    for b in range(31, -1, -1):
        bit = imin if b == 31 else jnp.int32(1 << b)
        hmask = jnp.int32(-(1 << b))
        cand = prefix | bit
        match = ((uk ^ cand) & hmask) == 0
        c = jnp.sum(match.astype(jnp.int32), axis=-1, keepdims=True)
        take = (count_above + c) >= k_sel
        prefix = jnp.where(take, cand, prefix)
        count_above = jnp.where(take, count_above, count_above + c)

    t_key = prefix ^ imin
    gt = key > t_key
    eq = key == t_key
    needed = k_sel - count_above

    idx = jax.lax.broadcasted_iota(jnp.int32, (B, S), 1)
    lo = jnp.zeros((B, 1), jnp.int32)
    hi = jnp.full((B, 1), S - 1, jnp.int32)
    for _ in range(12):
        mid = (lo + hi) // 2
        cnt = jnp.sum((eq & (idx <= mid)).astype(jnp.int32),
                      axis=-1, keepdims=True)
        ok = cnt >= needed
        hi = jnp.where(ok, mid, hi)
        lo = jnp.where(ok, lo, mid + 1)

    return (gt | (eq & (idx <= lo))).astype(jnp.int32)


def _fused_kernel(q_ref, k_ref, wqt_ref, wkt_ref, hw_ref, t_ref,
                  score_ref, mask_ref, qp_ref, kp_ref):
    b = pl.program_id(0)
    ti = pl.program_id(1)
    si = pl.program_id(2)

    @pl.when((b == 0) & (ti == 0) & (si == 0))
    def _init():
        score_ref[...] = jnp.zeros_like(score_ref)

    @pl.when(ti == 0)
    def _proj_q():
        qp_ref[si] = jax.lax.dot(
            q_ref[0], wqt_ref[...], precision=_PREC,
            preferred_element_type=jnp.float32)

    @pl.when(si == 0)
    def _proj_k():
        kp_ref[...] = jax.lax.dot(
            k_ref[0], wkt_ref[...], precision=_PREC,
            preferred_element_type=jnp.float32)

    qp = qp_ref[si]
    acc = jnp.zeros((1, TS), jnp.float32)
    for h in range(NH):
        p = jax.lax.dot_general(
            kp_ref[:, h * D:(h + 1) * D], qp[:, h * D:(h + 1) * D],
            (((1,), (1,)), ((), ())),
            precision=_PREC, preferred_element_type=jnp.float32)
        acc = acc + jnp.sum(jnp.maximum(p, 0.0), axis=0,
                            keepdims=True) * hw_ref[0, h]
    score_ref[pl.ds(b, 1), pl.ds(si * TS, TS)] += acc

    @pl.when((b == B - 1) & (ti == NT - 1) & (si == NS - 1))
    def _finish():
        scaled = score_ref[...] * jnp.exp(-t_ref[0, 0])
        score_ref[...] = scaled
        mask_ref[...] = _topk_mask(scaled)


def kernel(query_states, key_states, Wq, Wk, head_weights, temperature_param):
    hw = head_weights.reshape(1, NH).astype(jnp.float32)
    temp = temperature_param.reshape(1, 1).astype(jnp.float32)

    scores, mask_i32 = pl.pallas_call(
        _fused_kernel,
        grid=(B, NT, NS),
        in_specs=[
            pl.BlockSpec((1, TS, H), lambda b, ti, si: (b, si, 0)),
            pl.BlockSpec((1, TT, H), lambda b, ti, si: (b, ti, 0)),
            pl.BlockSpec((H, H), lambda b, ti, si: (0, 0)),
            pl.BlockSpec((H, H), lambda b, ti, si: (0, 0)),
            pl.BlockSpec((1, NH), lambda b, ti, si: (0, 0)),
            pl.BlockSpec((1, 1), lambda b, ti, si: (0, 0)),
        ],
        out_specs=[
            pl.BlockSpec((B, S), lambda b, ti, si: (0, 0)),
            pl.BlockSpec((B, S), lambda b, ti, si: (0, 0)),
        ],
        out_shape=[
            jax.ShapeDtypeStruct((B, S), jnp.float32),
            jax.ShapeDtypeStruct((B, S), jnp.int32),
        ],
        scratch_shapes=[
            pltpu.VMEM((NS, TS, H), jnp.float32),
            pltpu.VMEM((TT, H), jnp.float32),
        ],
    )(query_states, key_states, Wq.T, Wk.T, hw, temp)

    return (mask_i32.astype(jnp.bool_), scores)


# speedup vs baseline: 1.5097x; 1.0774x over previous
import jax
import jax.numpy as jnp
from jax.experimental import pallas as pl
from jax.experimental.pallas import tpu as pltpu

B, S, H = 2, 4096, 768
NH = 8
D = H // NH
TOPK = min(2048, S)

TS = 1024
TT = 1024
NS = S // TS
NT = S // TT

_PREC = jax.lax.Precision.DEFAULT


def _topk_mask(scores):
    i = jax.lax.bitcast_convert_type(scores, jnp.int32)
    key = jnp.where(i >= 0, i, i ^ jnp.int32(0x7FFFFFFF))
    imin = jnp.int32(-2147483648)
    uk = key ^ imin

    k_sel = jnp.int32(TOPK)
    prefix = jnp.zeros((B, 1), jnp.int32)
    count_above = jnp.zeros((B, 1), jnp.int32)
    for b in range(31, -1, -1):
        bit = imin if b == 31 else jnp.int32(1 << b)
        hmask = jnp.int32(-(1 << b))
        cand = prefix | bit
        match = ((uk ^ cand) & hmask) == 0
        c = jnp.sum(match.astype(jnp.int32), axis=-1, keepdims=True)
        take = (count_above + c) >= k_sel
        prefix = jnp.where(take, cand, prefix)
        count_above = jnp.where(take, count_above, count_above + c)

    t_key = prefix ^ imin
    gt = key > t_key
    eq = key == t_key
    needed = k_sel - count_above

    idx = jax.lax.broadcasted_iota(jnp.int32, (B, S), 1)
    lo = jnp.zeros((B, 1), jnp.int32)
    hi = jnp.full((B, 1), S - 1, jnp.int32)
    for _ in range(12):
        mid = (lo + hi) // 2
        cnt = jnp.sum((eq & (idx <= mid)).astype(jnp.int32),
                      axis=-1, keepdims=True)
        ok = cnt >= needed
        hi = jnp.where(ok, mid, hi)
        lo = jnp.where(ok, lo, mid + 1)

    return (gt | (eq & (idx <= lo))).astype(jnp.int32)


def _fused_kernel(q_ref, k_ref, wqt_ref, wkt_ref, hw_ref, t_ref,
                  score_ref, mask_ref, qp_ref, kp_ref):
    b = pl.program_id(0)
    ti = pl.program_id(1)
    si = pl.program_id(2)

    @pl.when((b == 0) & (ti == 0) & (si == 0))
    def _init():
        score_ref[...] = jnp.zeros_like(score_ref)

    @pl.when(ti == 0)
    def _proj_q():
        qp_ref[si] = jax.lax.dot(
            q_ref[0], wqt_ref[...], precision=_PREC,
            preferred_element_type=jnp.float32)

    @pl.when(si == 0)
    def _proj_k():
        kp_ref[...] = jax.lax.dot(
            k_ref[0], wkt_ref[...], precision=_PREC,
            preferred_element_type=jnp.float32)

    qp = qp_ref[si]
    acc = jnp.zeros((1, TS), jnp.float32)
    for h in range(NH):
        p = jax.lax.dot_general(
            kp_ref[:, h * D:(h + 1) * D], qp[:, h * D:(h + 1) * D],
            (((1,), (1,)), ((), ())),
            precision=_PREC, preferred_element_type=jnp.float32)
        acc = acc + jnp.sum(jnp.maximum(p, 0.0), axis=0,
                            keepdims=True) * hw_ref[0, h]
    score_ref[pl.ds(b, 1), pl.ds(si * TS, TS)] += acc

    @pl.when((b == B - 1) & (ti == NT - 1) & (si == NS - 1))
    def _finish():
        scaled = score_ref[...] * jnp.exp(-t_ref[0, 0])
        score_ref[...] = scaled
        mask_ref[...] = _topk_mask(scaled)


def kernel(query_states, key_states, Wq, Wk, head_weights, temperature_param):
    hw = head_weights.reshape(1, NH).astype(jnp.float32)
    temp = temperature_param.reshape(1, 1).astype(jnp.float32)

    scores, mask_i32 = pl.pallas_call(
        _fused_kernel,
        grid=(B, NT, NS),
        in_specs=[
            pl.BlockSpec((1, TS, H),
                         lambda b, ti, si: (b, jnp.where(ti == 0, si, NS - 1), 0)),
            pl.BlockSpec((1, TT, H), lambda b, ti, si: (b, ti, 0)),
            pl.BlockSpec((H, H), lambda b, ti, si: (0, 0)),
            pl.BlockSpec((H, H), lambda b, ti, si: (0, 0)),
            pl.BlockSpec((1, NH), lambda b, ti, si: (0, 0)),
            pl.BlockSpec((1, 1), lambda b, ti, si: (0, 0)),
        ],
        out_specs=[
            pl.BlockSpec((B, S), lambda b, ti, si: (0, 0)),
            pl.BlockSpec((B, S), lambda b, ti, si: (0, 0)),
        ],
        out_shape=[
            jax.ShapeDtypeStruct((B, S), jnp.float32),
            jax.ShapeDtypeStruct((B, S), jnp.int32),
        ],
        scratch_shapes=[
            pltpu.VMEM((NS, TS, H), jnp.float32),
            pltpu.VMEM((TT, H), jnp.float32),
        ],
    )(query_states, key_states, Wq.T, Wk.T, hw, temp)

    return (mask_i32.astype(jnp.bool_), scores)
